# Initial kernel scaffold; baseline (speedup 1.0000x reference)
import jax
import jax.numpy as jnp
from jax.experimental import pallas as pl


def kernel(input):
    raise NotImplementedError("write your pallas kernel here")


# speedup vs baseline: 1.1168x; 1.1168x over previous
import jax
import jax.numpy as jnp
from jax.experimental import pallas as pl

N = 20000
NPAD = 20480
ROWS = 160
B = 8
CONF_THRES = 0.25
IOU_THRES = 0.45
MAX_DET = 100
K_CAND = 300
KPAD = 384
MAX_WH = 7680.0
LO0 = 1048575999
HI0 = 1065353216
BS_ITERS = 25
PAD_KEY = -2147483648


def _prep_kernel(pred_ref, keys_ref, data_ref):
    p = pred_ref[0]
    cx = p[0:1, :]
    cy = p[1:2, :]
    w = p[2:3, :]
    h = p[3:4, :]
    obj = p[4:5, :]
    cls = p[5:85, :]
    scs = cls * obj
    conf = jnp.max(scs, axis=0, keepdims=True)
    cidx = jax.lax.broadcasted_iota(jnp.int32, (80, N), 0)
    j = jnp.min(jnp.where(scs == conf, cidx, 10000), axis=0, keepdims=True)
    valid = (obj > CONF_THRES) & (conf > CONF_THRES)
    smask = jnp.where(valid, conf, -1.0)
    key = jax.lax.bitcast_convert_type(smask, jnp.int32)

    x1 = cx - w * 0.5
    y1 = cy - h * 0.5
    x2 = cx + w * 0.5
    y2 = cy + h * 0.5

    keys_ref[0, :, pl.ds(0, N)] = key
    keys_ref[0, :, pl.ds(N, NPAD - N)] = jnp.full((1, NPAD - N), PAD_KEY, jnp.int32)

    zrow = jnp.zeros((1, N), jnp.float32)
    data = jnp.concatenate(
        [x1, y1, x2, y2, smask, j.astype(jnp.float32), zrow, zrow], axis=0
    )
    data_ref[0, :, pl.ds(0, N)] = data
    data_ref[0, :, pl.ds(N, NPAD - N)] = jnp.zeros((8, NPAD - N), jnp.float32)


def _exclusive_rank(mask):
    x = mask.astype(jnp.float32)
    ic = x
    for d in (1, 2, 4, 8, 16, 32, 64):
        shifted = jnp.concatenate(
            [jnp.zeros((B, ROWS, d), jnp.float32), ic[:, :, : 128 - d]], axis=2
        )
        ic = ic + shifted
    rowtot = ic[:, :, 127]
    upper = (
        jax.lax.broadcasted_iota(jnp.int32, (ROWS, ROWS), 0)
        < jax.lax.broadcasted_iota(jnp.int32, (ROWS, ROWS), 1)
    ).astype(jnp.float32)
    rowpre = jax.lax.dot_general(
        rowtot, upper, (((1,), (0,)), ((), ())),
        preferred_element_type=jnp.float32,
        precision=jax.lax.Precision.HIGHEST,
    )
    excl = ic - x + rowpre[:, :, None]
    return excl.astype(jnp.int32)


def _nms_kernel(keys_ref, data_ref, det_ref, num_ref):
    keys = keys_ref[...]
    data = data_ref[...]

    lo = jnp.full((B, 1, 1), LO0, jnp.int32)
    hi = jnp.full((B, 1, 1), HI0, jnp.int32)

    def bs_body(t, lh):
        lo, hi = lh
        mid = lo + jax.lax.shift_right_arithmetic(hi - lo, 1)
        cnt = jnp.sum((keys >= mid).astype(jnp.int32), axis=(1, 2), keepdims=True)
        ge = cnt >= K_CAND
        return jnp.where(ge, mid, lo), jnp.where(ge, hi, mid)

    lo, hi = jax.lax.fori_loop(0, BS_ITERS, bs_body, (lo, hi))
    kstar = lo

    gt = keys > kstar
    eq = keys == kstar
    c1 = jnp.sum(gt.astype(jnp.int32), axis=(1, 2), keepdims=True)
    need = K_CAND - c1
    eqr = _exclusive_rank(eq)
    sel = gt | (eq & (eqr < need))
    dst = _exclusive_rank(sel)
    dstm = jnp.where(sel, dst, -1)

    s_iota3 = jax.lax.broadcasted_iota(jnp.int32, (1, KPAD, 1), 1)

    cand = jnp.zeros((B, KPAD, 8), jnp.float32)
    for r in range(ROWS):
        dch = data[:, :, r * 128:(r + 1) * 128]
        dstch = dstm[:, r:r + 1, :]
        e = (dstch == s_iota3).astype(jnp.float32)
        cand = cand + jax.lax.dot_general(
            e, dch, (((2,), (2,)), ((0,), (0,))),
            preferred_element_type=jnp.float32,
            precision=jax.lax.Precision.HIGHEST,
        )

    score = cand[:, :, 4]
    off = cand[:, :, 5] * MAX_WH
    ox1 = cand[:, :, 0] + off
    oy1 = cand[:, :, 1] + off
    ox2 = cand[:, :, 2] + off
    oy2 = cand[:, :, 3] + off
    area = (ox2 - ox1) * (oy2 - oy1)
    ltx = jnp.maximum(ox1[:, :, None], ox1[:, None, :])
    rbx = jnp.minimum(ox2[:, :, None], ox2[:, None, :])
    wx = jnp.maximum(rbx - ltx, 0.0)
    lty = jnp.maximum(oy1[:, :, None], oy1[:, None, :])
    rby = jnp.minimum(oy2[:, :, None], oy2[:, None, :])
    wy = jnp.maximum(rby - lty, 0.0)
    inter = wx * wy
    iou = inter / (area[:, :, None] + area[:, None, :] - inter + 1e-9)

    siota2 = jax.lax.broadcasted_iota(jnp.int32, (B, KPAD), 1)

    def nms_body(i, state):
        sc, cnt = state
        bestval = jnp.max(sc, axis=1, keepdims=True)
        best = jnp.min(jnp.where(sc == bestval, siota2, KPAD), axis=1, keepdims=True)
        onehot = (siota2 == best).astype(jnp.float32)
        ok = bestval > 0.0
        row = jax.lax.dot_general(
            onehot, iou, (((1,), (1,)), ((0,), (0,))),
            preferred_element_type=jnp.float32,
            precision=jax.lax.Precision.HIGHEST,
        )
        sup = (row > IOU_THRES) | (onehot > 0.5)
        sc = jnp.where(sup, -1.0, sc)
        drow = jax.lax.dot_general(
            onehot, cand, (((1,), (1,)), ((0,), (0,))),
            preferred_element_type=jnp.float32,
            precision=jax.lax.Precision.HIGHEST,
        )
        drow = drow * ok.astype(jnp.float32)
        det_ref[:, pl.ds(i, 1), :] = drow[:, None, :]
        return sc, cnt + ok.astype(jnp.int32)

    _, cnt = jax.lax.fori_loop(
        0, MAX_DET, nms_body, (score, jnp.zeros((B, 1), jnp.int32))
    )
    num_ref[...] = jnp.broadcast_to(cnt, (B, 128))


def kernel(input):
    predT = jnp.transpose(input, (0, 2, 1))
    keys_a, data = pl.pallas_call(
        _prep_kernel,
        grid=(B,),
        in_specs=[pl.BlockSpec((1, 85, N), lambda i: (i, 0, 0))],
        out_specs=[
            pl.BlockSpec((1, 1, NPAD), lambda i: (i, 0, 0)),
            pl.BlockSpec((1, 8, NPAD), lambda i: (i, 0, 0)),
        ],
        out_shape=[
            jax.ShapeDtypeStruct((B, 1, NPAD), jnp.int32),
            jax.ShapeDtypeStruct((B, 8, NPAD), jnp.float32),
        ],
    )(predT)
    keys = keys_a.reshape(B, ROWS, 128)
    det8, num = pl.pallas_call(
        _nms_kernel,
        out_shape=[
            jax.ShapeDtypeStruct((B, MAX_DET, 8), jnp.float32),
            jax.ShapeDtypeStruct((B, 128), jnp.int32),
        ],
    )(keys, data)
    return det8[:, :, :6], num[:, 0]


# speedup vs baseline: 2.3607x; 2.1139x over previous
import jax
import jax.numpy as jnp
from jax.experimental import pallas as pl

N = 20000
NPAD = 20480
ROWS = 160
B = 8
CONF_THRES = 0.25
IOU_THRES = 0.45
MAX_DET = 100
K_CAND = 300
KPAD = 384
MAX_WH = 7680.0
LO0 = 1048575999
HI0 = 1065353216
BS_ITERS = 25
PAD_KEY = -2147483648


def _prep_kernel(pred_ref, keys_ref, data_ref):
    p = pred_ref[0]
    cx = p[0:1, :]
    cy = p[1:2, :]
    w = p[2:3, :]
    h = p[3:4, :]
    obj = p[4:5, :]
    cls = p[5:85, :]
    scs = cls * obj
    conf = jnp.max(scs, axis=0, keepdims=True)
    cidx = jax.lax.broadcasted_iota(jnp.int32, (80, N), 0)
    j = jnp.min(jnp.where(scs == conf, cidx, 10000), axis=0, keepdims=True)
    valid = (obj > CONF_THRES) & (conf > CONF_THRES)
    smask = jnp.where(valid, conf, -1.0)
    key = jax.lax.bitcast_convert_type(smask, jnp.int32)

    x1 = cx - w * 0.5
    y1 = cy - h * 0.5
    x2 = cx + w * 0.5
    y2 = cy + h * 0.5

    keys_ref[0, :, pl.ds(0, N)] = key
    keys_ref[0, :, pl.ds(N, NPAD - N)] = jnp.full((1, NPAD - N), PAD_KEY, jnp.int32)

    zrow = jnp.zeros((1, N), jnp.float32)
    data = jnp.concatenate(
        [x1, y1, x2, y2, smask, j.astype(jnp.float32), zrow, zrow], axis=0
    )
    data_ref[0, :, pl.ds(0, N)] = data
    data_ref[0, :, pl.ds(N, NPAD - N)] = jnp.zeros((8, NPAD - N), jnp.float32)


def _exclusive_rank(mask):
    x = mask.astype(jnp.float32)
    ic = x
    for d in (1, 2, 4, 8, 16, 32, 64):
        shifted = jnp.concatenate(
            [jnp.zeros((B, ROWS, d), jnp.float32), ic[:, :, : 128 - d]], axis=2
        )
        ic = ic + shifted
    rowtot = ic[:, :, 127]
    upper = (
        jax.lax.broadcasted_iota(jnp.int32, (ROWS, ROWS), 0)
        < jax.lax.broadcasted_iota(jnp.int32, (ROWS, ROWS), 1)
    ).astype(jnp.float32)
    rowpre = jax.lax.dot_general(
        rowtot, upper, (((1,), (0,)), ((), ())),
        preferred_element_type=jnp.float32,
        precision=jax.lax.Precision.HIGHEST,
    )
    excl = ic - x + rowpre[:, :, None]
    return excl.astype(jnp.int32)


def _nms_kernel(keys_ref, data_ref, det_ref, num_ref):
    keys = keys_ref[...]
    data = data_ref[...]

    lo = jnp.full((B, 1, 1), LO0, jnp.int32)
    hi = jnp.full((B, 1, 1), HI0, jnp.int32)

    def bs_body(t, lh):
        lo, hi = lh
        mid = lo + jax.lax.shift_right_arithmetic(hi - lo, 1)
        cnt = jnp.sum((keys >= mid).astype(jnp.int32), axis=(1, 2), keepdims=True)
        ge = cnt >= K_CAND
        return jnp.where(ge, mid, lo), jnp.where(ge, hi, mid)

    lo, hi = jax.lax.fori_loop(0, BS_ITERS, bs_body, (lo, hi))
    kstar = lo

    gt = keys > kstar
    eq = keys == kstar
    c1 = jnp.sum(gt.astype(jnp.int32), axis=(1, 2), keepdims=True)
    need = K_CAND - c1
    eqr = _exclusive_rank(eq)
    sel = gt | (eq & (eqr < need))
    dst = _exclusive_rank(sel)
    dstm = jnp.where(sel, dst, -1)

    s_iota3 = jax.lax.broadcasted_iota(jnp.int32, (1, KPAD, 1), 1)
    dhi = data.astype(jnp.bfloat16)
    r1 = data - dhi.astype(jnp.float32)
    dmid = r1.astype(jnp.bfloat16)
    dlo = (r1 - dmid.astype(jnp.float32)).astype(jnp.bfloat16)
    dsplit = jnp.concatenate([dhi, dmid, dlo], axis=1)

    cand24 = jnp.zeros((B, KPAD, 24), jnp.float32)
    for r in range(ROWS):
        dch = dsplit[:, :, r * 128:(r + 1) * 128]
        dstch = dstm[:, r:r + 1, :]
        e = (dstch == s_iota3).astype(jnp.float32).astype(jnp.bfloat16)
        cand24 = cand24 + jax.lax.dot_general(
            e, dch, (((2,), (2,)), ((0,), (0,))),
            preferred_element_type=jnp.float32,
        )
    cand = cand24[:, :, 0:8] + cand24[:, :, 8:16] + cand24[:, :, 16:24]

    score = cand[:, :, 4]
    off = cand[:, :, 5] * MAX_WH
    ox1 = cand[:, :, 0] + off
    oy1 = cand[:, :, 1] + off
    ox2 = cand[:, :, 2] + off
    oy2 = cand[:, :, 3] + off
    area = (ox2 - ox1) * (oy2 - oy1)
    ltx = jnp.maximum(ox1[:, :, None], ox1[:, None, :])
    rbx = jnp.minimum(ox2[:, :, None], ox2[:, None, :])
    wx = jnp.maximum(rbx - ltx, 0.0)
    lty = jnp.maximum(oy1[:, :, None], oy1[:, None, :])
    rby = jnp.minimum(oy2[:, :, None], oy2[:, None, :])
    wy = jnp.maximum(rby - lty, 0.0)
    inter = wx * wy
    iou = inter / (area[:, :, None] + area[:, None, :] - inter + 1e-9)
    supm = (iou > IOU_THRES).astype(jnp.float32).astype(jnp.bfloat16)

    siota2 = jax.lax.broadcasted_iota(jnp.int32, (B, KPAD), 1)

    def nms_body(i, state):
        sc, cnt = state
        bestval = jnp.max(sc, axis=1, keepdims=True)
        best = jnp.min(jnp.where(sc == bestval, siota2, KPAD), axis=1, keepdims=True)
        onehot = (siota2 == best).astype(jnp.float32)
        onehot_b = onehot.astype(jnp.bfloat16)
        ok = bestval > 0.0
        row = jax.lax.dot_general(
            onehot_b, supm, (((1,), (1,)), ((0,), (0,))),
            preferred_element_type=jnp.float32,
        )
        sup = (row > 0.5) | (onehot > 0.5)
        sc = jnp.where(sup, -1.0, sc)
        drow = jax.lax.dot_general(
            onehot, cand, (((1,), (1,)), ((0,), (0,))),
            preferred_element_type=jnp.float32,
            precision=jax.lax.Precision.HIGHEST,
        )
        drow = drow * ok.astype(jnp.float32)
        det_ref[:, pl.ds(i, 1), :] = drow[:, None, :]
        return sc, cnt + ok.astype(jnp.int32)

    _, cnt = jax.lax.fori_loop(
        0, MAX_DET, nms_body, (score, jnp.zeros((B, 1), jnp.int32))
    )
    num_ref[...] = jnp.broadcast_to(cnt, (B, 128))


def kernel(input):
    predT = jnp.transpose(input, (0, 2, 1))
    keys_a, data = pl.pallas_call(
        _prep_kernel,
        grid=(B,),
        in_specs=[pl.BlockSpec((1, 85, N), lambda i: (i, 0, 0))],
        out_specs=[
            pl.BlockSpec((1, 1, NPAD), lambda i: (i, 0, 0)),
            pl.BlockSpec((1, 8, NPAD), lambda i: (i, 0, 0)),
        ],
        out_shape=[
            jax.ShapeDtypeStruct((B, 1, NPAD), jnp.int32),
            jax.ShapeDtypeStruct((B, 8, NPAD), jnp.float32),
        ],
    )(predT)
    keys = keys_a.reshape(B, ROWS, 128)
    det8, num = pl.pallas_call(
        _nms_kernel,
        out_shape=[
            jax.ShapeDtypeStruct((B, MAX_DET, 8), jnp.float32),
            jax.ShapeDtypeStruct((B, 128), jnp.int32),
        ],
    )(keys, data)
    return det8[:, :, :6], num[:, 0]
